# Initial kernel scaffold; baseline (speedup 1.0000x reference)
import jax
import jax.numpy as jnp
from jax.experimental import pallas as pl


def kernel(node_x, net_x, edge_index_sink, edge_weight_sink, edge_index_source, batch, W_ne1, b_ne1, W_ne2, b_ne2, W_nete, b_nete, Wc_node, bc_node, Wc_net, bc_net, W_f1n, b_f1n, W_f2n, b_f2n, W_f1t, b_f1t, W_f2t, b_f2t, W_fin, b_fin):
    raise NotImplementedError("write your pallas kernel here")


# speedup vs baseline: 6.1741x; 6.1741x over previous
import functools

import jax
import jax.numpy as jnp
from jax import lax
from jax.experimental import pallas as pl
from jax.experimental.pallas import tpu as pltpu
from jax.experimental.pallas import tpu_sc as plsc

N_NODES = 10000
N_NETS = 10000
EMB = 128
L = 3

NC = 2
NS = 16
NW = NC * NS
CHUNK = 128
SUB = 1
TS_SINK = 80
TS_NET = 82
EP_SINK = NW * TS_SINK * CHUNK
EP_NET = NW * TS_NET * CHUNK
ACC_ROWS = 10240

_f32 = jnp.float32
_i32 = jnp.int32


def _leaky(x):
    return jnp.where(x >= 0, x, 0.1 * x)


@functools.lru_cache(maxsize=None)
def _make_sc_pass(ts):
    mesh = plsc.VectorSubcoreMesh(core_axis_name="c", subcore_axis_name="s",
                                  num_cores=NC, num_subcores=NS)

    scratch = [
        pltpu.VMEM((SUB, 128), _i32),
        pltpu.VMEM((SUB, 128), _i32),
        pltpu.VMEM((SUB, 128), _i32),
        pltpu.VMEM((SUB, 128), _i32),
        pltpu.VMEM((CHUNK,), _f32),
        pltpu.VMEM((CHUNK,), _f32),
        pltpu.VMEM((CHUNK, EMB), _f32),
        pltpu.VMEM((CHUNK, EMB), _f32),
        pltpu.VMEM_SHARED((ACC_ROWS, EMB), _f32),
        pltpu.SemaphoreType.DMA,
        pltpu.SemaphoreType.DMA,
        pltpu.SemaphoreType.DMA,
        pltpu.SemaphoreType.DMA,
    ]

    def body(tbl, g_h, s_h, w_h, out, gv0, gv1, sv0, sv1, wv0, wv1,
             rows0, rows1, accum, gsem0, gsem1, ssem0, ssem1):
        bufs = ((gv0, sv0, wv0, rows0, gsem0, ssem0),
                (gv1, sv1, wv1, rows1, gsem1, ssem1))

        c = lax.axis_index("c")
        s = lax.axis_index("s")
        w = c * NS + s

        zeros16 = jnp.zeros((16,), _f32)

        def _zb(e, carry):
            for q in range(EMB // 16):
                rows0[e, pl.ds(q * 16, 16)] = zeros16
            return carry

        lax.fori_loop(0, CHUNK, _zb, 0, unroll=8)
        for z in range(5):
            pltpu.sync_copy(rows0, accum.at[pl.ds(s * 640 + z * 128, 128)])
        plsc.subcore_barrier()

        dnums = lax.GatherDimensionNumbers(
            offset_dims=(), collapsed_slice_dims=(0,), start_index_map=(0,))
        jidx = [jnp.full((16, 1), j, _i32) for j in range(16)]

        def _scale(rows, wv):
            def gb(g, carry):
                w16 = wv[pl.ds(g * 16, 16)]
                for j in range(16):
                    spl = lax.gather(
                        w16, jidx[j], dnums, (1,),
                        mode=lax.GatherScatterMode.PROMISE_IN_BOUNDS)
                    e = g * 16 + j
                    for q in range(EMB // 16):
                        sl = pl.ds(q * 16, 16)
                        rows[e, sl] = rows[e, sl] * spl
                return carry
            lax.fori_loop(0, CHUNK // 16, gb, 0)

        def _issue(chunk_idx, buf):
            gv, sv, wv, rows, gsem, _ = buf
            pltpu.sync_copy(g_h.at[pl.ds(chunk_idx * SUB, SUB)], gv)
            pltpu.sync_copy(s_h.at[pl.ds(chunk_idx * SUB, SUB)], sv)
            pltpu.sync_copy(w_h.at[pl.ds(chunk_idx * CHUNK, CHUNK)], wv)
            for jj in range(SUB):
                pltpu.async_copy(tbl.at[gv.at[jj]],
                                 rows.at[pl.ds(jj * 128, 128)], gsem)

        def _process(buf):
            gv, sv, wv, rows, gsem, ssem = buf
            pltpu.make_async_copy(tbl.at[pl.ds(0, CHUNK)], rows, gsem).wait()
            _scale(rows, wv)
            for jj in range(SUB):
                pltpu.async_copy(rows.at[pl.ds(jj * 128, 128)],
                                 accum.at[sv.at[jj]], ssem, add=True)

        def _drain_scatter(buf):
            rows, ssem = buf[3], buf[5]
            pltpu.make_async_copy(rows, accum.at[pl.ds(0, CHUNK)], ssem).wait()

        base = w * ts
        _issue(base + 0, bufs[0])
        _issue(base + 1, bufs[1])

        def pair(k2, carry):
            _process(bufs[0])
            _process(bufs[1])

            @pl.when(k2 + 1 < ts // 2)
            def _():
                _drain_scatter(bufs[0])
                _issue(base + 2 * k2 + 2, bufs[0])
                _drain_scatter(bufs[1])
                _issue(base + 2 * k2 + 3, bufs[1])
            return carry

        lax.fori_loop(0, ts // 2, pair, 0)
        _drain_scatter(bufs[0])
        _drain_scatter(bufs[1])

        plsc.subcore_barrier()
        pltpu.sync_copy(accum.at[pl.ds(s * 640, 640)],
                        out.at[c, pl.ds(s * 640, 640)])

    return pl.kernel(
        body,
        out_type=jax.ShapeDtypeStruct((NC, ACC_ROWS, EMB), _f32),
        mesh=mesh,
        scratch_types=scratch,
    )


def _sc_net_pass(*args):
    return _make_sc_pass(TS_NET)(*args)


def _sc_back_pass(*args):
    return _make_sc_pass(TS_SINK)(*args)


def _pad_edges(g, sidx, w, e_pad):
    e0 = g.shape[0]
    ar = jnp.arange(e_pad - e0, dtype=_i32)
    g = jnp.concatenate([g, ar % N_NODES])
    sidx = jnp.concatenate([sidx, ar % N_NETS])
    w = jnp.concatenate([w, jnp.zeros((e_pad - e0,), _f32)])
    return g.reshape(e_pad // 128, 128), sidx.reshape(e_pad // 128, 128), w


BR = 1000
_GRID = N_NODES // BR
_PREC = jax.lax.Precision.DEFAULT


def _dot(a, b):
    return jnp.dot(a, b, preferred_element_type=_f32, precision=_PREC)


def _dot_hi(a, b):
    return jnp.dot(a, b, preferred_element_type=_f32,
                   precision=jax.lax.Precision.HIGHEST)


def _full(shape):
    return pl.BlockSpec(shape, lambda i: (0,) * len(shape))


def _rows(shape):
    return pl.BlockSpec(shape, lambda i: (i,) + (0,) * (len(shape) - 1))


def _enc_node_body(x, w1, b1, w2, b2, o):
    y = _leaky(_dot(x[...], w1[...]) + b1[...])
    o[...] = _leaky(_dot(y, w2[...]) + b2[...])


def _enc_net_body(x, w, b, o):
    o[...] = _leaky(_dot(x[...], w[...]) + b[...])


def _layer_body(base, p0, p1, w, b, o):
    t = base[...] + p0[...] + p1[...]
    o[...] = _leaky(_dot(t, w[...]) + b[...])


def _final_node_body(rep, wn, b1, w2, b2, w3, b3, o):
    t = _leaky(_dot(rep[...], wn[...]) + b1[...])
    u = _leaky(_dot_hi(t, w2[...]) + b2[...])
    o[...] = _dot_hi(u, w3[...]) + b3[...]


def _final_net_body(rep, wn, b1, w2, b2, o):
    t = _leaky(_dot(rep[...], wn[...]) + b1[...])
    o[...] = jnp.abs(_leaky(_dot_hi(t, w2[...]) + b2[...]))


def _enc_node(x, w1, b1, w2, b2):
    return pl.pallas_call(
        _enc_node_body,
        grid=(_GRID,),
        in_specs=[_rows((BR, EMB)), _full((EMB, EMB)), _full((1, EMB)),
                  _full((EMB, EMB)), _full((1, EMB))],
        out_specs=_rows((BR, EMB)),
        out_shape=jax.ShapeDtypeStruct((N_NODES, EMB), _f32),
    )(x, w1, b1.reshape(1, EMB), w2, b2.reshape(1, EMB))


def _enc_net(x, w, b):
    return pl.pallas_call(
        _enc_net_body,
        grid=(_GRID,),
        in_specs=[_rows((BR, 16)), _full((16, EMB)), _full((1, EMB))],
        out_specs=_rows((BR, EMB)),
        out_shape=jax.ShapeDtypeStruct((N_NETS, EMB), _f32),
    )(x, w, b.reshape(1, EMB))


def _layer_update(base, p0, p1, w, b):
    return pl.pallas_call(
        _layer_body,
        grid=(_GRID,),
        in_specs=[_rows((BR, EMB)), _rows((BR, EMB)), _rows((BR, EMB)),
                  _full((EMB, EMB)), _full((1, EMB))],
        out_specs=_rows((BR, EMB)),
        out_shape=jax.ShapeDtypeStruct((N_NODES, EMB), _f32),
    )(base, p0, p1, w, b.reshape(1, EMB))


def _final_node(hs, w1, b1, w2, b2, w3, b3):
    rep = jnp.concatenate(hs, axis=1)
    return pl.pallas_call(
        _final_node_body,
        grid=(_GRID,),
        in_specs=[_rows((BR, 4 * EMB)),
                  _full((4 * EMB, 256)), _full((1, 256)),
                  _full((256, 32)), _full((1, 32)),
                  _full((32, 1)), _full((1, 1))],
        out_specs=_rows((BR, 1)),
        out_shape=jax.ShapeDtypeStruct((N_NODES, 1), _f32),
    )(rep, w1, b1.reshape(1, 256),
      w2, b2.reshape(1, 32), w3, b3.reshape(1, 1))


def _final_net(hs, w1, b1, w2, b2):
    rep = jnp.concatenate(hs, axis=1)
    return pl.pallas_call(
        _final_net_body,
        grid=(_GRID,),
        in_specs=[_rows((BR, 4 * EMB)),
                  _full((4 * EMB, 64)), _full((1, 64)),
                  _full((64, 8)), _full((1, 8))],
        out_specs=_rows((BR, 8)),
        out_shape=jax.ShapeDtypeStruct((N_NETS, 8), _f32),
    )(rep, w1, b1.reshape(1, 64), w2, b2.reshape(1, 8))


def kernel(node_x, net_x, edge_index_sink, edge_weight_sink, edge_index_source,
           batch, W_ne1, b_ne1, W_ne2, b_ne2, W_nete, b_nete,
           Wc_node, bc_node, Wc_net, bc_net,
           W_f1n, b_f1n, W_f2n, b_f2n, W_f1t, b_f1t, W_f2t, b_f2t,
           W_fin, b_fin):
    keep = jax.random.bernoulli(
        jax.random.key(42), 0.6, (edge_index_sink.shape[1],)).astype(_f32)
    ew = edge_weight_sink * keep

    src_n = edge_index_sink[0].astype(_i32)
    dst_n = edge_index_sink[1].astype(_i32)
    s_src = edge_index_source[0].astype(_i32)
    s_dst = edge_index_source[1].astype(_i32)

    gA, sA, wA = _pad_edges(
        jnp.concatenate([src_n, s_src]),
        jnp.concatenate([dst_n, s_dst]),
        jnp.concatenate([ew, jnp.ones((s_src.shape[0],), _f32)]),
        EP_NET)
    gC, sC, wC = _pad_edges(dst_n, src_n, ew, EP_SINK)

    h = _enc_node(node_x, W_ne1, b_ne1, W_ne2, b_ne2)
    hn = _enc_net(net_x, W_nete, b_nete)

    hs = [h]
    hns = [hn]
    for l in range(L):
        p = _sc_net_pass(h, gA, sA, wA)
        hn = _layer_update(hn, p[0, :N_NETS], p[1, :N_NETS],
                           Wc_net[l], bc_net[l])
        q = _sc_back_pass(hn, gC, sC, wC)
        h = _layer_update(h, q[0, :N_NODES], q[1, :N_NODES],
                          Wc_node[l], bc_node[l])
        hs.append(h)
        hns.append(hn)

    node_out = _final_node(hs, W_f1n, b_f1n, W_f2n, b_f2n, W_fin, b_fin)
    net_out = _final_net(hns, W_f1t, b_f1t, W_f2t, b_f2t)
    return node_out, net_out


# speedup vs baseline: 8.9907x; 1.4562x over previous
import functools

import jax
import jax.numpy as jnp
from jax import lax
from jax.experimental import pallas as pl
from jax.experimental.pallas import tpu as pltpu
from jax.experimental.pallas import tpu_sc as plsc

N_NODES = 10000
N_NETS = 10000
EMB = 128
L = 3

NC = 2
NS = 16
NW = NC * NS
CHUNK = 64
NBUF = 4
NIDX = 8
TS_SINK = 160
TS_NET = 168
EP_SINK = NW * TS_SINK * CHUNK
EP_NET = NW * TS_NET * CHUNK
ACC_ROWS = 10240

_f32 = jnp.float32
_i32 = jnp.int32


def _leaky(x):
    return jnp.where(x >= 0, x, 0.1 * x)


@functools.lru_cache(maxsize=None)
def _make_sc_pass(ts):
    mesh = plsc.VectorSubcoreMesh(core_axis_name="c", subcore_axis_name="s",
                                  num_cores=NC, num_subcores=NS)

    scratch = (
        [pltpu.VMEM((NIDX, CHUNK), _i32),
         pltpu.VMEM((NIDX, CHUNK), _i32),
         pltpu.VMEM((NIDX, CHUNK), _f32)]
        + [pltpu.VMEM((CHUNK, EMB), _f32) for _ in range(NBUF)]
        + [pltpu.VMEM_SHARED((ACC_ROWS, EMB), _f32)]
        + [pltpu.SemaphoreType.DMA] * (2 * NBUF + NIDX)
    )

    def body(tbl, g_h, s_h, w_h, out, gv, sv, wv, *rest):
        rows = rest[:NBUF]
        accum = rest[NBUF]
        gsem = rest[NBUF + 1: NBUF + 1 + NBUF]
        ssem = rest[NBUF + 1 + NBUF: NBUF + 1 + 2 * NBUF]
        isem = rest[NBUF + 1 + 2 * NBUF:]

        c = lax.axis_index("c")
        s = lax.axis_index("s")
        w = c * NS + s

        zeros16 = jnp.zeros((16,), _f32)

        def _zb(e, carry):
            for q in range(EMB // 16):
                rows[0][e, pl.ds(q * 16, 16)] = zeros16
            return carry

        lax.fori_loop(0, CHUNK, _zb, 0, unroll=8)
        for z in range(640 // CHUNK):
            pltpu.sync_copy(rows[0],
                            accum.at[pl.ds(s * 640 + z * CHUNK, CHUNK)])
        plsc.subcore_barrier()

        dnums = lax.GatherDimensionNumbers(
            offset_dims=(), collapsed_slice_dims=(0,), start_index_map=(0,))
        jidx = [jnp.full((16, 1), j, _i32) for j in range(16)]

        def _scale(b4, b8):
            def gb(g, carry):
                w16 = wv[b8, pl.ds(g * 16, 16)]
                for j in range(16):
                    spl = lax.gather(
                        w16, jidx[j], dnums, (1,),
                        mode=lax.GatherScatterMode.PROMISE_IN_BOUNDS)
                    e = g * 16 + j
                    for q in range(EMB // 16):
                        sl = pl.ds(q * 16, 16)
                        rows[b4][e, sl] = rows[b4][e, sl] * spl
                return carry
            lax.fori_loop(0, CHUNK // 16, gb, 0)

        def _issue_idx(chunk_idx, b8):
            pltpu.async_copy(g_h.at[chunk_idx], gv.at[b8], isem[b8])
            pltpu.async_copy(s_h.at[chunk_idx], sv.at[b8], isem[b8])
            pltpu.async_copy(w_h.at[chunk_idx], wv.at[b8], isem[b8])

        def _wait_idx(b8):
            pltpu.make_async_copy(g_h.at[0], gv.at[b8], isem[b8]).wait()
            pltpu.make_async_copy(s_h.at[0], sv.at[b8], isem[b8]).wait()
            pltpu.make_async_copy(w_h.at[0], wv.at[b8], isem[b8]).wait()

        def _issue_gather(b4, b8):
            pltpu.async_copy(tbl.at[gv.at[b8]], rows[b4], gsem[b4])

        def _wait_gather(b4):
            pltpu.make_async_copy(tbl.at[pl.ds(0, CHUNK)], rows[b4],
                                  gsem[b4]).wait()

        def _start_scatter(b4, b8):
            pltpu.async_copy(rows[b4], accum.at[sv.at[b8]], ssem[b4],
                             add=True)

        def _drain_scatter(b4):
            pltpu.make_async_copy(rows[b4], accum.at[pl.ds(0, CHUNK)],
                                  ssem[b4]).wait()

        base = w * ts
        for j in range(4):
            _issue_idx(base + j, j)
        for j in range(2):
            _wait_idx(j)
            _issue_gather(j, j)

        def octet(kk, carry):
            for b in range(8):
                k = kk * 8 + b
                b4 = b % NBUF
                _wait_gather(b4)
                _scale(b4, b)
                _start_scatter(b4, b)
                if b >= 2:
                    _drain_scatter((b + 2) % NBUF)
                else:
                    @pl.when(kk > 0)
                    def _():
                        _drain_scatter((b + 2) % NBUF)

                @pl.when(k + 2 < ts)
                def _():
                    _wait_idx((b + 2) % NIDX)
                    _issue_gather((b + 2) % NBUF, (b + 2) % NIDX)

                @pl.when(k + 4 < ts)
                def _():
                    _issue_idx(base + k + 4, (b + 4) % NIDX)
            return carry

        lax.fori_loop(0, ts // 8, octet, 0)
        _drain_scatter((ts - 2) % NBUF)
        _drain_scatter((ts - 1) % NBUF)

        plsc.subcore_barrier()
        pltpu.sync_copy(accum.at[pl.ds(s * 640, 640)],
                        out.at[c, pl.ds(s * 640, 640)])

    return pl.kernel(
        body,
        out_type=jax.ShapeDtypeStruct((NC, ACC_ROWS, EMB), _f32),
        mesh=mesh,
        scratch_types=scratch,
    )


def _sc_net_pass(*args):
    return _make_sc_pass(TS_NET)(*args)


def _sc_back_pass(*args):
    return _make_sc_pass(TS_SINK)(*args)


def _pad_edges(g, sidx, w, e_pad):
    e0 = g.shape[0]
    ar = jnp.arange(e_pad - e0, dtype=_i32)
    g = jnp.concatenate([g, ar % N_NODES])
    sidx = jnp.concatenate([sidx, ar % N_NETS])
    w = jnp.concatenate([w, jnp.zeros((e_pad - e0,), _f32)])
    return (g.reshape(e_pad // CHUNK, CHUNK),
            sidx.reshape(e_pad // CHUNK, CHUNK),
            w.reshape(e_pad // CHUNK, CHUNK))


BR = 1000
_GRID = N_NODES // BR
_PREC = jax.lax.Precision.DEFAULT


def _dot(a, b):
    return jnp.dot(a, b, preferred_element_type=_f32, precision=_PREC)


def _dot_hi(a, b):
    return jnp.dot(a, b, preferred_element_type=_f32,
                   precision=jax.lax.Precision.HIGHEST)


def _full(shape):
    return pl.BlockSpec(shape, lambda i: (0,) * len(shape))


def _rows(shape):
    return pl.BlockSpec(shape, lambda i: (i,) + (0,) * (len(shape) - 1))


def _enc_node_body(x, w1, b1, w2, b2, o):
    y = _leaky(_dot(x[...], w1[...]) + b1[...])
    o[...] = _leaky(_dot(y, w2[...]) + b2[...])


def _enc_net_body(x, w, b, o):
    o[...] = _leaky(_dot(x[...], w[...]) + b[...])


def _layer_body(base, p0, p1, w, b, o):
    t = base[...] + p0[...] + p1[...]
    o[...] = _leaky(_dot(t, w[...]) + b[...])


def _final_node_body(rep, wn, b1, w2, b2, w3, b3, o):
    t = _leaky(_dot(rep[...], wn[...]) + b1[...])
    u = _leaky(_dot_hi(t, w2[...]) + b2[...])
    o[...] = _dot_hi(u, w3[...]) + b3[...]


def _final_net_body(rep, wn, b1, w2, b2, o):
    t = _leaky(_dot(rep[...], wn[...]) + b1[...])
    o[...] = jnp.abs(_leaky(_dot_hi(t, w2[...]) + b2[...]))


def _enc_node(x, w1, b1, w2, b2):
    return pl.pallas_call(
        _enc_node_body,
        grid=(_GRID,),
        in_specs=[_rows((BR, EMB)), _full((EMB, EMB)), _full((1, EMB)),
                  _full((EMB, EMB)), _full((1, EMB))],
        out_specs=_rows((BR, EMB)),
        out_shape=jax.ShapeDtypeStruct((N_NODES, EMB), _f32),
    )(x, w1, b1.reshape(1, EMB), w2, b2.reshape(1, EMB))


def _enc_net(x, w, b):
    return pl.pallas_call(
        _enc_net_body,
        grid=(_GRID,),
        in_specs=[_rows((BR, 16)), _full((16, EMB)), _full((1, EMB))],
        out_specs=_rows((BR, EMB)),
        out_shape=jax.ShapeDtypeStruct((N_NETS, EMB), _f32),
    )(x, w, b.reshape(1, EMB))


def _layer_update(base, p0, p1, w, b):
    return pl.pallas_call(
        _layer_body,
        grid=(_GRID,),
        in_specs=[_rows((BR, EMB)), _rows((BR, EMB)), _rows((BR, EMB)),
                  _full((EMB, EMB)), _full((1, EMB))],
        out_specs=_rows((BR, EMB)),
        out_shape=jax.ShapeDtypeStruct((N_NODES, EMB), _f32),
    )(base, p0, p1, w, b.reshape(1, EMB))


def _final_node(hs, w1, b1, w2, b2, w3, b3):
    rep = jnp.concatenate(hs, axis=1)
    return pl.pallas_call(
        _final_node_body,
        grid=(_GRID,),
        in_specs=[_rows((BR, 4 * EMB)),
                  _full((4 * EMB, 256)), _full((1, 256)),
                  _full((256, 32)), _full((1, 32)),
                  _full((32, 1)), _full((1, 1))],
        out_specs=_rows((BR, 1)),
        out_shape=jax.ShapeDtypeStruct((N_NODES, 1), _f32),
    )(rep, w1, b1.reshape(1, 256),
      w2, b2.reshape(1, 32), w3, b3.reshape(1, 1))


def _final_net(hs, w1, b1, w2, b2):
    rep = jnp.concatenate(hs, axis=1)
    return pl.pallas_call(
        _final_net_body,
        grid=(_GRID,),
        in_specs=[_rows((BR, 4 * EMB)),
                  _full((4 * EMB, 64)), _full((1, 64)),
                  _full((64, 8)), _full((1, 8))],
        out_specs=_rows((BR, 8)),
        out_shape=jax.ShapeDtypeStruct((N_NETS, 8), _f32),
    )(rep, w1, b1.reshape(1, 64), w2, b2.reshape(1, 8))


def kernel(node_x, net_x, edge_index_sink, edge_weight_sink, edge_index_source,
           batch, W_ne1, b_ne1, W_ne2, b_ne2, W_nete, b_nete,
           Wc_node, bc_node, Wc_net, bc_net,
           W_f1n, b_f1n, W_f2n, b_f2n, W_f1t, b_f1t, W_f2t, b_f2t,
           W_fin, b_fin):
    keep = jax.random.bernoulli(
        jax.random.key(42), 0.6, (edge_index_sink.shape[1],)).astype(_f32)
    ew = edge_weight_sink * keep

    src_n = edge_index_sink[0].astype(_i32)
    dst_n = edge_index_sink[1].astype(_i32)
    s_src = edge_index_source[0].astype(_i32)
    s_dst = edge_index_source[1].astype(_i32)

    gA, sA, wA = _pad_edges(
        jnp.concatenate([src_n, s_src]),
        jnp.concatenate([dst_n, s_dst]),
        jnp.concatenate([ew, jnp.ones((s_src.shape[0],), _f32)]),
        EP_NET)
    gC, sC, wC = _pad_edges(dst_n, src_n, ew, EP_SINK)

    h = _enc_node(node_x, W_ne1, b_ne1, W_ne2, b_ne2)
    hn = _enc_net(net_x, W_nete, b_nete)

    hs = [h]
    hns = [hn]
    for l in range(L):
        p = _sc_net_pass(h, gA, sA, wA)
        hn = _layer_update(hn, p[0, :N_NETS], p[1, :N_NETS],
                           Wc_net[l], bc_net[l])
        q = _sc_back_pass(hn, gC, sC, wC)
        h = _layer_update(h, q[0, :N_NODES], q[1, :N_NODES],
                          Wc_node[l], bc_node[l])
        hs.append(h)
        hns.append(hn)

    node_out = _final_node(hs, W_f1n, b_f1n, W_f2n, b_f2n, W_fin, b_fin)
    net_out = _final_net(hns, W_f1t, b_f1t, W_f2t, b_f2t)
    return node_out, net_out


# speedup vs baseline: 9.5529x; 1.0625x over previous
import functools

import jax
import jax.numpy as jnp
from jax import lax
from jax.experimental import pallas as pl
from jax.experimental.pallas import tpu as pltpu
from jax.experimental.pallas import tpu_sc as plsc

N_NODES = 10000
N_NETS = 10000
EMB = 128
L = 3

NC = 2
NS = 16
NW = NC * NS
CHUNK = 64
NBUF = 5
NIDX = 10
TS_SINK = 160
TS_NET = 170
EP_SINK = NW * TS_SINK * CHUNK
EP_NET = NW * TS_NET * CHUNK
ACC_ROWS = 10240

_f32 = jnp.float32
_i32 = jnp.int32


def _leaky(x):
    return jnp.where(x >= 0, x, 0.1 * x)


@functools.lru_cache(maxsize=None)
def _make_sc_pass(ts):
    mesh = plsc.VectorSubcoreMesh(core_axis_name="c", subcore_axis_name="s",
                                  num_cores=NC, num_subcores=NS)

    scratch = (
        [pltpu.VMEM((NIDX, CHUNK), _i32),
         pltpu.VMEM((NIDX, CHUNK), _i32),
         pltpu.VMEM((NIDX, CHUNK), _f32)]
        + [pltpu.VMEM((CHUNK, EMB), _f32) for _ in range(NBUF)]
        + [pltpu.VMEM_SHARED((ACC_ROWS, EMB), _f32)]
        + [pltpu.SemaphoreType.DMA] * (2 * NBUF + NIDX)
    )

    def body(tbl, g_h, s_h, w_h, out, gv, sv, wv, *rest):
        rows = rest[:NBUF]
        accum = rest[NBUF]
        gsem = rest[NBUF + 1: NBUF + 1 + NBUF]
        ssem = rest[NBUF + 1 + NBUF: NBUF + 1 + 2 * NBUF]
        isem = rest[NBUF + 1 + 2 * NBUF:]

        c = lax.axis_index("c")
        s = lax.axis_index("s")
        w = c * NS + s

        zeros16 = jnp.zeros((16,), _f32)

        def _zb(e, carry):
            for q in range(EMB // 16):
                rows[0][e, pl.ds(q * 16, 16)] = zeros16
            return carry

        lax.fori_loop(0, CHUNK, _zb, 0, unroll=8)
        for z in range(640 // CHUNK):
            pltpu.sync_copy(rows[0],
                            accum.at[pl.ds(s * 640 + z * CHUNK, CHUNK)])
        plsc.subcore_barrier()

        dnums = lax.GatherDimensionNumbers(
            offset_dims=(), collapsed_slice_dims=(0,), start_index_map=(0,))
        jidx = [jnp.full((16, 1), j, _i32) for j in range(16)]

        def _scale(b4, b8):
            def gb(g, carry):
                w16 = wv[b8, pl.ds(g * 16, 16)]
                for j in range(16):
                    spl = lax.gather(
                        w16, jidx[j], dnums, (1,),
                        mode=lax.GatherScatterMode.PROMISE_IN_BOUNDS)
                    e = g * 16 + j
                    for q in range(EMB // 16):
                        sl = pl.ds(q * 16, 16)
                        rows[b4][e, sl] = rows[b4][e, sl] * spl
                return carry
            lax.fori_loop(0, CHUNK // 16, gb, 0)

        def _issue_idx(chunk_idx, b8):
            pltpu.async_copy(g_h.at[chunk_idx], gv.at[b8], isem[b8])
            pltpu.async_copy(s_h.at[chunk_idx], sv.at[b8], isem[b8])
            pltpu.async_copy(w_h.at[chunk_idx], wv.at[b8], isem[b8])

        def _wait_idx(b8):
            pltpu.make_async_copy(g_h.at[0], gv.at[b8], isem[b8]).wait()
            pltpu.make_async_copy(s_h.at[0], sv.at[b8], isem[b8]).wait()
            pltpu.make_async_copy(w_h.at[0], wv.at[b8], isem[b8]).wait()

        def _issue_gather(b4, b8):
            pltpu.async_copy(tbl.at[gv.at[b8]], rows[b4], gsem[b4])

        def _wait_gather(b4):
            pltpu.make_async_copy(tbl.at[pl.ds(0, CHUNK)], rows[b4],
                                  gsem[b4]).wait()

        def _start_scatter(b4, b8):
            pltpu.async_copy(rows[b4], accum.at[sv.at[b8]], ssem[b4],
                             add=True)

        def _drain_scatter(b4):
            pltpu.make_async_copy(rows[b4], accum.at[pl.ds(0, CHUNK)],
                                  ssem[b4]).wait()

        base = w * ts
        for j in range(NBUF):
            _issue_idx(base + j, j)
        for j in range(3):
            _wait_idx(j)
            _issue_gather(j, j)

        def dectet(kk, carry):
            for b in range(NIDX):
                k = kk * NIDX + b
                b5 = b % NBUF
                _wait_gather(b5)
                _scale(b5, b)
                _start_scatter(b5, b)
                if b >= 2:
                    _drain_scatter((b + 3) % NBUF)
                else:
                    @pl.when(kk > 0)
                    def _():
                        _drain_scatter((b + 3) % NBUF)

                @pl.when(k + 3 < ts)
                def _():
                    _wait_idx((b + 3) % NIDX)
                    _issue_gather((b + 3) % NBUF, (b + 3) % NIDX)

                @pl.when(k + 5 < ts)
                def _():
                    _issue_idx(base + k + 5, (b + 5) % NIDX)
            return carry

        lax.fori_loop(0, ts // NIDX, dectet, 0)
        _drain_scatter((ts - 2) % NBUF)
        _drain_scatter((ts - 1) % NBUF)

        plsc.subcore_barrier()
        pltpu.sync_copy(accum.at[pl.ds(s * 640, 640)],
                        out.at[c, pl.ds(s * 640, 640)])

    return pl.kernel(
        body,
        out_type=jax.ShapeDtypeStruct((NC, ACC_ROWS, EMB), _f32),
        mesh=mesh,
        scratch_types=scratch,
    )


def _sc_net_pass(*args):
    return _make_sc_pass(TS_NET)(*args)


def _sc_back_pass(*args):
    return _make_sc_pass(TS_SINK)(*args)


def _pad_edges(g, sidx, w, e_pad):
    e0 = g.shape[0]
    ar = jnp.arange(e_pad - e0, dtype=_i32)
    g = jnp.concatenate([g, ar % N_NODES])
    sidx = jnp.concatenate([sidx, ar % N_NETS])
    w = jnp.concatenate([w, jnp.zeros((e_pad - e0,), _f32)])
    return (g.reshape(e_pad // CHUNK, CHUNK),
            sidx.reshape(e_pad // CHUNK, CHUNK),
            w.reshape(e_pad // CHUNK, CHUNK))


BR = 1000
_GRID = N_NODES // BR
_PREC = jax.lax.Precision.DEFAULT


def _dot(a, b):
    return jnp.dot(a, b, preferred_element_type=_f32, precision=_PREC)


def _dot_hi(a, b):
    return jnp.dot(a, b, preferred_element_type=_f32,
                   precision=jax.lax.Precision.HIGHEST)


def _full(shape):
    return pl.BlockSpec(shape, lambda i: (0,) * len(shape))


def _rows(shape):
    return pl.BlockSpec(shape, lambda i: (i,) + (0,) * (len(shape) - 1))


def _enc_body(x, nx, w1, b1, w2, b2, wn, bn, oh, on):
    y = _leaky(_dot(x[...], w1[...]) + b1[...])
    oh[...] = _leaky(_dot(y, w2[...]) + b2[...])
    on[...] = _leaky(_dot(nx[...], wn[...]) + bn[...])


def _layer_body(base, p0, p1, w, b, o):
    t = base[...] + p0[...] + p1[...]
    o[...] = _leaky(_dot(t, w[...]) + b[...])


def _final_body(repn, rept, w1n, b1n, w2n, b2n, w3n, b3n,
                w1t, b1t, w2t, b2t, onode, onet):
    t = _leaky(_dot(repn[...], w1n[...]) + b1n[...])
    u = _leaky(_dot_hi(t, w2n[...]) + b2n[...])
    onode[...] = _dot_hi(u, w3n[...]) + b3n[...]
    v = _leaky(_dot(rept[...], w1t[...]) + b1t[...])
    onet[...] = jnp.abs(_leaky(_dot_hi(v, w2t[...]) + b2t[...]))


def _encoders(x, nx, w1, b1, w2, b2, wn, bn):
    return pl.pallas_call(
        _enc_body,
        grid=(_GRID,),
        in_specs=[_rows((BR, EMB)), _rows((BR, 16)),
                  _full((EMB, EMB)), _full((1, EMB)),
                  _full((EMB, EMB)), _full((1, EMB)),
                  _full((16, EMB)), _full((1, EMB))],
        out_specs=[_rows((BR, EMB)), _rows((BR, EMB))],
        out_shape=[jax.ShapeDtypeStruct((N_NODES, EMB), _f32),
                   jax.ShapeDtypeStruct((N_NETS, EMB), _f32)],
    )(x, nx, w1, b1.reshape(1, EMB), w2, b2.reshape(1, EMB),
      wn, bn.reshape(1, EMB))


def _layer_update(base, p0, p1, w, b):
    return pl.pallas_call(
        _layer_body,
        grid=(_GRID,),
        in_specs=[_rows((BR, EMB)), _rows((BR, EMB)), _rows((BR, EMB)),
                  _full((EMB, EMB)), _full((1, EMB))],
        out_specs=_rows((BR, EMB)),
        out_shape=jax.ShapeDtypeStruct((N_NODES, EMB), _f32),
    )(base, p0, p1, w, b.reshape(1, EMB))


def _finals(hs, hns, w1n, b1n, w2n, b2n, w3n, b3n, w1t, b1t, w2t, b2t):
    repn = jnp.concatenate(hs, axis=1)
    rept = jnp.concatenate(hns, axis=1)
    return pl.pallas_call(
        _final_body,
        grid=(_GRID,),
        in_specs=[_rows((BR, 4 * EMB)), _rows((BR, 4 * EMB)),
                  _full((4 * EMB, 256)), _full((1, 256)),
                  _full((256, 32)), _full((1, 32)),
                  _full((32, 1)), _full((1, 1)),
                  _full((4 * EMB, 64)), _full((1, 64)),
                  _full((64, 8)), _full((1, 8))],
        out_specs=[_rows((BR, 1)), _rows((BR, 8))],
        out_shape=[jax.ShapeDtypeStruct((N_NODES, 1), _f32),
                   jax.ShapeDtypeStruct((N_NETS, 8), _f32)],
    )(repn, rept, w1n, b1n.reshape(1, 256), w2n, b2n.reshape(1, 32),
      w3n, b3n.reshape(1, 1), w1t, b1t.reshape(1, 64), w2t, b2t.reshape(1, 8))


def kernel(node_x, net_x, edge_index_sink, edge_weight_sink, edge_index_source,
           batch, W_ne1, b_ne1, W_ne2, b_ne2, W_nete, b_nete,
           Wc_node, bc_node, Wc_net, bc_net,
           W_f1n, b_f1n, W_f2n, b_f2n, W_f1t, b_f1t, W_f2t, b_f2t,
           W_fin, b_fin):
    keep = jax.random.bernoulli(
        jax.random.key(42), 0.6, (edge_index_sink.shape[1],)).astype(_f32)
    ew = edge_weight_sink * keep

    src_n = edge_index_sink[0].astype(_i32)
    dst_n = edge_index_sink[1].astype(_i32)
    s_src = edge_index_source[0].astype(_i32)
    s_dst = edge_index_source[1].astype(_i32)

    gA, sA, wA = _pad_edges(
        jnp.concatenate([src_n, s_src]),
        jnp.concatenate([dst_n, s_dst]),
        jnp.concatenate([ew, jnp.ones((s_src.shape[0],), _f32)]),
        EP_NET)
    gC, sC, wC = _pad_edges(dst_n, src_n, ew, EP_SINK)

    h, hn = _encoders(node_x, net_x, W_ne1, b_ne1, W_ne2, b_ne2,
                      W_nete, b_nete)

    hs = [h]
    hns = [hn]
    for l in range(L):
        p = _sc_net_pass(h, gA, sA, wA)
        hn = _layer_update(hn, p[0, :N_NETS], p[1, :N_NETS],
                           Wc_net[l], bc_net[l])
        q = _sc_back_pass(hn, gC, sC, wC)
        h = _layer_update(h, q[0, :N_NODES], q[1, :N_NODES],
                          Wc_node[l], bc_node[l])
        hs.append(h)
        hns.append(hn)

    node_out, net_out = _finals(hs, hns, W_f1n, b_f1n, W_f2n, b_f2n,
                                W_fin, b_fin, W_f1t, b_f1t, W_f2t, b_f2t)
    return node_out, net_out
